# baseline (speedup 1.0000x reference)
import functools

import jax
import jax.numpy as jnp
from jax import lax
from jax.experimental import pallas as pl
from jax.experimental.pallas import tpu as pltpu
from jax.experimental.pallas import tpu_sc as plsc

NUM_USER = 10000
NUM_MOVIES = 5000
EMB = 32
BATCH = 16384

_SLOPE = 0.01

_NC, _NS, _L = 2, 16, 16
_NW = _NC * _NS
_BPW = BATCH // _NW
_ICH = 128
_NCH = _BPW // _ICH


def _leaky(z):
    return jnp.where(z >= 0, z, _SLOPE * z)


def _movie_pass_body(adj1_ref, adj2_ref, rhs_ref, prev_ref, W_ref, b_ref,
                     out_ref):
    dot = lambda x, y: jax.lax.dot_general(
        x, y, (((1,), (0,)), ((), ())), preferred_element_type=jnp.float32)
    rhs = rhs_ref[...]
    W = W_ref[...]
    b = b_ref[...]
    bm = adj1_ref.shape[0]
    for half, adj_ref in enumerate((adj1_ref, adj2_ref)):
        p = dot(adj_ref[...], rhs)
        prev = prev_ref[pl.ds(half * bm, bm), :]
        z = (dot(p, W) + b) + (dot(prev, W) + b)
        out_ref[pl.ds(half * bm, bm), :] = _leaky(z)


def _movie_pass(adj, rhs, prev, W, b, bm):
    n = adj.shape[0]
    k = adj.shape[1]
    grid = ((n + 2 * bm - 1) // (2 * bm),)
    return pl.pallas_call(
        _movie_pass_body,
        grid=grid,
        in_specs=[
            pl.BlockSpec((bm, k), lambda i: (2 * i, 0)),
            pl.BlockSpec((bm, k), lambda i: (2 * i + 1, 0)),
            pl.BlockSpec((k, EMB), lambda i: (0, 0)),
            pl.BlockSpec((2 * bm, EMB), lambda i: (i, 0)),
            pl.BlockSpec((EMB, EMB), lambda i: (0, 0)),
            pl.BlockSpec((1, EMB), lambda i: (0, 0)),
        ],
        out_specs=pl.BlockSpec((2 * bm, EMB), lambda i: (i, 0)),
        out_shape=jax.ShapeDtypeStruct((n, EMB), jnp.float32),
        compiler_params=pltpu.CompilerParams(
            vmem_limit_bytes=64 * 1024 * 1024),
    )(adj, adj, rhs, prev, W, b)


def _user_pass_body(adj1_ref, adj2_ref, mm_ref, u0_ref, uW_ref, ub_ref,
                    u1_ref, u2_ref):
    dot = lambda x, y: jax.lax.dot_general(
        x, y, (((1,), (0,)), ((), ())), preferred_element_type=jnp.float32)
    mm = mm_ref[...]
    b0, b1 = ub_ref[0], ub_ref[1]
    bm = adj1_ref.shape[0]
    for half, adj_ref in enumerate((adj1_ref, adj2_ref)):
        u0 = u0_ref[pl.ds(half * bm, bm), :]
        q = dot(adj_ref[...], mm)
        u1 = _leaky((dot(q[:, :EMB], uW_ref[0]) + b0)
                    + (dot(u0, uW_ref[0]) + b0))
        u2 = _leaky((dot(q[:, EMB:], uW_ref[1]) + b1)
                    + (dot(u1, uW_ref[1]) + b1))
        u1_ref[pl.ds(half * bm, bm), :] = u1
        u2_ref[pl.ds(half * bm, bm), :] = u2


def _user_pass(adj, mm, u0, uW, ub, bm):
    n = adj.shape[0]
    k = adj.shape[1]
    grid = ((n + 2 * bm - 1) // (2 * bm),)
    return pl.pallas_call(
        _user_pass_body,
        grid=grid,
        in_specs=[
            pl.BlockSpec((bm, k), lambda i: (2 * i, 0)),
            pl.BlockSpec((bm, k), lambda i: (2 * i + 1, 0)),
            pl.BlockSpec((k, 2 * EMB), lambda i: (0, 0)),
            pl.BlockSpec((2 * bm, EMB), lambda i: (i, 0)),
            pl.BlockSpec((2, EMB, EMB), lambda i: (0, 0, 0)),
            pl.BlockSpec((2, 1, EMB), lambda i: (0, 0, 0)),
        ],
        out_specs=[
            pl.BlockSpec((2 * bm, EMB), lambda i: (i, 0)),
            pl.BlockSpec((2 * bm, EMB), lambda i: (i, 0)),
        ],
        out_shape=[
            jax.ShapeDtypeStruct((n, EMB), jnp.float32),
            jax.ShapeDtypeStruct((n, EMB), jnp.float32),
        ],
        compiler_params=pltpu.CompilerParams(
            vmem_limit_bytes=64 * 1024 * 1024),
    )(adj, adj, mm, u0, uW, ub)


def _sc_interact_body(uid_hbm, mid_hbm, u0_hbm, u1_hbm, u2_hbm,
                      m0_hbm, m1_hbm, m2_hbm, wb_hbm, bvec_hbm, out_hbm,
                      idx_u, idx_m, gu0, gu1, gu2, gm0, gm1, gm2, wv, bv_ref,
                      outv, sem):
    wid = lax.axis_index("s") * _NC + lax.axis_index("c")
    pltpu.sync_copy(uid_hbm.at[wid], idx_u)
    pltpu.sync_copy(mid_hbm.at[wid], idx_m)
    pltpu.sync_copy(wb_hbm, wv)
    pltpu.sync_copy(bvec_hbm, bv_ref)
    copies = []
    for tbl, idxv, buf in ((u0_hbm, idx_u, gu0), (u1_hbm, idx_u, gu1),
                           (u2_hbm, idx_u, gu2), (m0_hbm, idx_m, gm0),
                           (m1_hbm, idx_m, gm1), (m2_hbm, idx_m, gm2)):
        for j in range(_NCH):
            copies.append(pltpu.async_copy(
                tbl.at[idxv.at[j]], buf.at[pl.ds(j * _ICH, _ICH)], sem))
    for cp in copies:
        cp.wait()

    wch = [wv[pl.ds(c * _L, _L)] for c in range(6)]
    bv = bv_ref[...]
    lane0 = lax.iota(jnp.int32, _L) == 0

    def body(i, carry):
        acc = bv
        for t, (gu, gm) in enumerate(((gu0, gm0), (gu1, gm1), (gu2, gm2))):
            for c in range(2):
                a = gu[i, pl.ds(c * _L, _L)]
                b = gm[i, pl.ds(c * _L, _L)]
                ab = a * b
                u = plsc.bitcast(ab, jnp.int32)
                u = (u + 0x7FFF + ((u >> 16) & 1)) & ~0xFFFF
                p = plsc.bitcast(u, jnp.float32)
                acc = acc + p * wch[2 * t + c]
        s = jnp.sum(acc, axis=0)
        plsc.store_scatter(outv, [jnp.full((_L,), i, jnp.int32)],
                           jnp.full((_L,), s, jnp.float32), mask=lane0)
        return carry

    lax.fori_loop(0, _BPW, body, 0)
    pltpu.sync_copy(outv, out_hbm.at[pl.ds(wid * _BPW, _BPW)])


@functools.lru_cache(maxsize=1)
def _sc_interact():
    mesh = plsc.VectorSubcoreMesh(core_axis_name="c", subcore_axis_name="s")
    return pl.kernel(
        _sc_interact_body,
        out_type=jax.ShapeDtypeStruct((BATCH,), jnp.float32),
        mesh=mesh,
        compiler_params=pltpu.CompilerParams(use_tc_tiling_on_sc=False,
                                             needs_layout_passes=False),
        scratch_types=[
            pltpu.VMEM((_NCH, _ICH), jnp.int32),
            pltpu.VMEM((_NCH, _ICH), jnp.int32),
            pltpu.VMEM((_BPW, EMB), jnp.float32),
            pltpu.VMEM((_BPW, EMB), jnp.float32),
            pltpu.VMEM((_BPW, EMB), jnp.float32),
            pltpu.VMEM((_BPW, EMB), jnp.float32),
            pltpu.VMEM((_BPW, EMB), jnp.float32),
            pltpu.VMEM((_BPW, EMB), jnp.float32),
            pltpu.VMEM((6 * _L,), jnp.float32),
            pltpu.VMEM((_L,), jnp.float32),
            pltpu.VMEM((_BPW,), jnp.float32),
            pltpu.SemaphoreType.DMA,
        ],
    )


def kernel(user_adj, movie_adj, user_id, movie_id, user_table, movie_table,
           user_W, user_b, movie_W, movie_b, out_W, out_b):
    u0 = user_table
    m0 = movie_table
    ub = user_b.reshape(2, 1, EMB)
    mb = movie_b.reshape(2, 1, EMB)
    w = out_W.reshape(1, 3 * EMB)

    m1 = _movie_pass(movie_adj, u0, m0, movie_W[0], mb[0], bm=256)
    mm = jnp.concatenate([m0, m1], axis=1)
    u1, u2 = _user_pass(user_adj, mm, u0, user_W, ub, bm=512)
    m2 = _movie_pass(movie_adj, u1, m1, movie_W[1], mb[1], bm=256)

    uid = user_id.astype(jnp.int32).reshape(_NW, _NCH, _ICH)
    mid = movie_id.astype(jnp.int32).reshape(_NW, _NCH, _ICH)
    wb = out_W.reshape(-1).astype(jnp.bfloat16).astype(jnp.float32)
    bvec = jnp.zeros((_L,), jnp.float32).at[0].set(out_b[0])
    return _sc_interact()(uid, mid, u0, u1, u2, m0, m1, m2, wb, bvec)


# speedup vs baseline: 1.0538x; 1.0538x over previous
import functools

import jax
import jax.numpy as jnp
from jax import lax
from jax.experimental import pallas as pl
from jax.experimental.pallas import tpu as pltpu
from jax.experimental.pallas import tpu_sc as plsc

NUM_USER = 10000
NUM_MOVIES = 5000
EMB = 32
BATCH = 16384

_SLOPE = 0.01

_NC, _NS, _L = 2, 16, 16
_NW = _NC * _NS
_BPW = BATCH // _NW
_ICH = 128
_NCH = _BPW // _ICH


def _leaky(z):
    return jnp.where(z >= 0, z, _SLOPE * z)


def _movie_pass_body(adj_ref, rhs_ref, prev_ref, W_ref, b_ref, out_ref):
    dot = lambda x, y: jax.lax.dot_general(
        x, y, (((1,), (0,)), ((), ())), preferred_element_type=jnp.float32)
    p = dot(adj_ref[...], rhs_ref[...])
    b = b_ref[...]
    z = (dot(p, W_ref[...]) + b) + (dot(prev_ref[...], W_ref[...]) + b)
    out_ref[...] = _leaky(z)


def _movie_pass(adj, rhs, prev, W, b, bm):
    n = adj.shape[0]
    k = adj.shape[1]
    grid = ((n + bm - 1) // bm,)
    return pl.pallas_call(
        _movie_pass_body,
        grid=grid,
        in_specs=[
            pl.BlockSpec((bm, k), lambda i: (i, 0)),
            pl.BlockSpec((k, EMB), lambda i: (0, 0)),
            pl.BlockSpec((bm, EMB), lambda i: (i, 0)),
            pl.BlockSpec((EMB, EMB), lambda i: (0, 0)),
            pl.BlockSpec((1, EMB), lambda i: (0, 0)),
        ],
        out_specs=pl.BlockSpec((bm, EMB), lambda i: (i, 0)),
        out_shape=jax.ShapeDtypeStruct((n, EMB), jnp.float32),
        compiler_params=pltpu.CompilerParams(
            vmem_limit_bytes=64 * 1024 * 1024),
    )(adj, rhs, prev, W, b)


def _user_pass_body(adj_ref, mm_ref, u0_ref, uW_ref, ub_ref,
                    u1_ref, u2_ref):
    dot = lambda x, y: jax.lax.dot_general(
        x, y, (((1,), (0,)), ((), ())), preferred_element_type=jnp.float32)
    a = adj_ref[...]
    u0 = u0_ref[...]
    q = dot(a, mm_ref[...])
    b0, b1 = ub_ref[0], ub_ref[1]
    u1 = _leaky((dot(q[:, :EMB], uW_ref[0]) + b0) + (dot(u0, uW_ref[0]) + b0))
    u2 = _leaky((dot(q[:, EMB:], uW_ref[1]) + b1) + (dot(u1, uW_ref[1]) + b1))
    u1_ref[...] = u1
    u2_ref[...] = u2


def _user_pass(adj, mm, u0, uW, ub, bm):
    n = adj.shape[0]
    k = adj.shape[1]
    grid = ((n + bm - 1) // bm,)
    blk = lambda r, c: pl.BlockSpec((r, c), lambda i: (i, 0))
    full = lambda shape: pl.BlockSpec(shape, lambda i: tuple(0 for _ in shape))
    return pl.pallas_call(
        _user_pass_body,
        grid=grid,
        in_specs=[
            blk(bm, k),
            full((k, 2 * EMB)),
            blk(bm, EMB),
            full((2, EMB, EMB)),
            full((2, 1, EMB)),
        ],
        out_specs=[
            pl.BlockSpec((bm, EMB), lambda i: (i, 0)),
            pl.BlockSpec((bm, EMB), lambda i: (i, 0)),
        ],
        out_shape=[
            jax.ShapeDtypeStruct((n, EMB), jnp.float32),
            jax.ShapeDtypeStruct((n, EMB), jnp.float32),
        ],
        compiler_params=pltpu.CompilerParams(
            vmem_limit_bytes=64 * 1024 * 1024),
    )(adj, mm, u0, uW, ub)


def _bf16_round(x):
    u = plsc.bitcast(x, jnp.int32)
    u = (u + 0x7FFF + ((u >> 16) & 1)) & ~0xFFFF
    return plsc.bitcast(u, jnp.float32)


def _interact_groups(tables, wch, outv, init_of_group, acc0):
    onehots = [(lax.iota(jnp.int32, _L) == j).astype(jnp.float32)
               for j in range(_L)]

    def body(g, carry):
        res = init_of_group(g)
        for j in range(_L):
            acc = acc0
            for gu, gm, w0 in tables:
                i = g * _L + j
                for c in range(2):
                    a = gu[i, pl.ds(c * _L, _L)]
                    b = gm[i, pl.ds(c * _L, _L)]
                    acc = acc + _bf16_round(a * b) * wch[w0 + c]
            s = jnp.sum(acc, axis=0)
            res = res + jnp.full((_L,), s, jnp.float32) * onehots[j]
        outv[pl.ds(g * _L, _L)] = res
        return carry

    lax.fori_loop(0, _BPW // _L, body, 0)


def _gather_all(pairs, sem):
    copies = []
    for tbl, idxv, buf in pairs:
        for j in range(_NCH):
            copies.append(pltpu.async_copy(
                tbl.at[idxv.at[j]], buf.at[pl.ds(j * _ICH, _ICH)], sem))
    for cp in copies:
        cp.wait()


def _sc_a_body(uid_hbm, mid_hbm, u0_hbm, u1_hbm, m0_hbm, m1_hbm,
               wb_hbm, bvec_hbm, out_hbm,
               idx_u, idx_m, gu0, gu1, gm0, gm1, wv, bv_ref, outv, sem):
    wid = lax.axis_index("s") * _NC + lax.axis_index("c")
    pltpu.sync_copy(uid_hbm.at[wid], idx_u)
    pltpu.sync_copy(mid_hbm.at[wid], idx_m)
    pltpu.sync_copy(wb_hbm, wv)
    pltpu.sync_copy(bvec_hbm, bv_ref)
    _gather_all(((u0_hbm, idx_u, gu0), (u1_hbm, idx_u, gu1),
                 (m0_hbm, idx_m, gm0), (m1_hbm, idx_m, gm1)), sem)
    wch = [wv[pl.ds(c * _L, _L)] for c in range(4)]
    bv = bv_ref[...]
    zero = jnp.zeros((_L,), jnp.float32)
    _interact_groups([(gu0, gm0, 0), (gu1, gm1, 2)], wch, outv,
                     lambda g: zero, bv)
    pltpu.sync_copy(outv, out_hbm.at[pl.ds(wid * _BPW, _BPW)])


def _sc_b_body(uid_hbm, mid_hbm, u2_hbm, m2_hbm, wb_hbm, part_hbm, out_hbm,
               idx_u, idx_m, gu2, gm2, wv, partv, outv, sem):
    wid = lax.axis_index("s") * _NC + lax.axis_index("c")
    pltpu.sync_copy(uid_hbm.at[wid], idx_u)
    pltpu.sync_copy(mid_hbm.at[wid], idx_m)
    pltpu.sync_copy(wb_hbm, wv)
    pltpu.sync_copy(part_hbm.at[pl.ds(wid * _BPW, _BPW)], partv)
    _gather_all(((u2_hbm, idx_u, gu2), (m2_hbm, idx_m, gm2)), sem)
    wch = [None, None, None, None,
           wv[pl.ds(4 * _L, _L)], wv[pl.ds(5 * _L, _L)]]
    zero = jnp.zeros((_L,), jnp.float32)
    _interact_groups([(gu2, gm2, 4)], wch, outv,
                     lambda g: partv[pl.ds(g * _L, _L)], zero)
    pltpu.sync_copy(outv, out_hbm.at[pl.ds(wid * _BPW, _BPW)])


def _sc_common(body, n_tables, n_w):
    mesh = plsc.VectorSubcoreMesh(core_axis_name="c", subcore_axis_name="s")
    scratch = [
        pltpu.VMEM((_NCH, _ICH), jnp.int32),
        pltpu.VMEM((_NCH, _ICH), jnp.int32),
    ] + [pltpu.VMEM((_BPW, EMB), jnp.float32)] * n_tables + [
        pltpu.VMEM((6 * _L,), jnp.float32),
    ] + n_w + [
        pltpu.VMEM((_BPW,), jnp.float32),
        pltpu.SemaphoreType.DMA,
    ]
    return pl.kernel(
        body,
        out_type=jax.ShapeDtypeStruct((BATCH,), jnp.float32),
        mesh=mesh,
        compiler_params=pltpu.CompilerParams(use_tc_tiling_on_sc=False,
                                             needs_layout_passes=False),
        scratch_types=scratch,
    )


@functools.lru_cache(maxsize=1)
def _sc_a():
    return _sc_common(_sc_a_body, 4, [pltpu.VMEM((_L,), jnp.float32)])


@functools.lru_cache(maxsize=1)
def _sc_b():
    return _sc_common(_sc_b_body, 2, [pltpu.VMEM((_BPW,), jnp.float32)])


def kernel(user_adj, movie_adj, user_id, movie_id, user_table, movie_table,
           user_W, user_b, movie_W, movie_b, out_W, out_b):
    u0 = user_table
    m0 = movie_table
    ub = user_b.reshape(2, 1, EMB)
    mb = movie_b.reshape(2, 1, EMB)
    w = out_W.reshape(1, 3 * EMB)

    m1 = _movie_pass(movie_adj, u0, m0, movie_W[0], mb[0], bm=512)
    mm = jnp.concatenate([m0, m1], axis=1)
    u1, u2 = _user_pass(user_adj, mm, u0, user_W, ub, bm=1024)
    m2 = _movie_pass(movie_adj, u1, m1, movie_W[1], mb[1], bm=512)

    uid = user_id.astype(jnp.int32).reshape(_NW, _NCH, _ICH)
    mid = movie_id.astype(jnp.int32).reshape(_NW, _NCH, _ICH)
    wb = out_W.reshape(-1).astype(jnp.bfloat16).astype(jnp.float32)
    bvec = jnp.zeros((_L,), jnp.float32).at[0].set(out_b[0])
    part = _sc_a()(uid, mid, u0, u1, m0, m1, wb, bvec)
    return _sc_b()(uid, mid, u2, m2, wb, part)
